# baseline (speedup 1.0000x reference)
import functools

import jax
import jax.numpy as jnp
from jax import lax
from jax.experimental import pallas as pl
from jax.experimental.pallas import tpu as pltpu
from jax.experimental.pallas import tpu_sc as plsc

DIM = 16
K = 10
B = 1024
N = 1000
NP = 1024
NLANE = 16
NCHUNK = 16
CLEN = NP // NCHUNK
JVALID = N - (NCHUNK - 1) * CLEN
NC = 2
NS = 16
NW = NC * NS
RPW = B // NW
CAND = NP + NLANE
BLK = 128


def _tc_scores_body(bp_ref, ipt_ref, v_ref, out_ref, cmax_ref, cmin_ref):
    bp = bp_ref[...]
    ipt = ipt_ref[...]
    vv = v_ref[...].astype(jnp.bfloat16).astype(jnp.float32)
    acc = jnp.zeros((BLK, NP), jnp.float32)
    for d in range(DIM):
        h = jnp.tanh(bp[:, d:d + 1] + ipt[d:d + 1, :])
        hq = h.astype(jnp.bfloat16).astype(jnp.float32)
        acc = acc + vv[0:1, d:d + 1] * hq
    out_ref[...] = acc

    inv = (lax.broadcasted_iota(jnp.int32, (1, 128), 1) % NCHUNK) == NCHUNK - 1
    neg = jnp.float32(-jnp.inf)
    pos = jnp.float32(jnp.inf)
    cols_t = [acc[:, k * 128:(k + 1) * 128] for k in range(8)]
    cols_b = list(cols_t)
    for k in range(5, 8):
        cols_t[k] = jnp.where(inv, neg, cols_t[k])
        cols_b[k] = jnp.where(inv, pos, cols_b[k])
    mx = functools.reduce(jnp.maximum, cols_t)
    mn = functools.reduce(jnp.minimum, cols_b)
    for w in (64, 32, 16):
        mx = jnp.maximum(mx[:, :w], mx[:, w:2 * w])
        mn = jnp.minimum(mn[:, :w], mn[:, w:2 * w])
    cmax_ref[...] = mx
    cmin_ref[...] = mn


@functools.cache
def _tc_scores(nrows):
    return pl.pallas_call(
        _tc_scores_body,
        grid=(nrows // BLK,),
        in_specs=[
            pl.BlockSpec((BLK, DIM), lambda i: (i, 0)),
            pl.BlockSpec((DIM, NP), lambda i: (0, 0)),
            pl.BlockSpec((1, DIM), lambda i: (0, 0)),
        ],
        out_specs=[pl.BlockSpec((BLK, NP), lambda i: (i, 0)),
                   pl.BlockSpec((BLK, NCHUNK), lambda i: (i, 0)),
                   pl.BlockSpec((BLK, NCHUNK), lambda i: (i, 0))],
        out_shape=[jax.ShapeDtypeStruct((nrows, NP), jnp.float32),
                   jax.ShapeDtypeStruct((nrows, NCHUNK), jnp.float32),
                   jax.ShapeDtypeStruct((nrows, NCHUNK), jnp.float32)],
    )


def _sc_topk_body(rpw, scores_hbm, cmax_hbm, cmin_hbm, pos_hbm, neg_hbm,
                  rows_v, cmax_v, cmin_v, tk_v, ti_v, bk_v, bi_v, pos_v, neg_v):
    wid = lax.axis_index("s") * NC + lax.axis_index("c")
    base = wid * rpw
    pltpu.sync_copy(scores_hbm.at[pl.ds(base, rpw)], rows_v)
    pltpu.sync_copy(cmax_hbm.at[pl.ds(base, rpw)], cmax_v)
    pltpu.sync_copy(cmin_hbm.at[pl.ds(base, rpw)], cmin_v)

    lanes = lax.iota(jnp.int32, NLANE)
    lane_last = lanes == (NCHUNK - 1)
    neg_inf = jnp.full((NLANE,), -jnp.inf, jnp.float32)
    pos_inf = jnp.full((NLANE,), jnp.inf, jnp.float32)

    def row_body(r, _):
        tt = jnp.sort(cmax_v[r, :])[jnp.zeros((NLANE,), jnp.int32)]
        tb = jnp.sort(cmin_v[r, :])[jnp.full((NLANE,), NLANE - 1, jnp.int32)]

        def cpt(j, carry):
            ct, cb = carry
            vv = rows_v[r, pl.ds(j * NLANE, NLANE)]
            valid = jnp.logical_not(jnp.logical_and(lane_last, j >= JVALID))
            idx = lanes * CLEN + j
            mt = jnp.logical_and(vv >= tt, valid)
            mb = jnp.logical_and(vv <= tb, valid)
            plsc.store_compressed(tk_v.at[pl.ds(ct, NLANE)], vv, mask=mt)
            plsc.store_compressed(ti_v.at[pl.ds(ct, NLANE)], idx, mask=mt)
            plsc.store_compressed(bk_v.at[pl.ds(cb, NLANE)], vv, mask=mb)
            plsc.store_compressed(bi_v.at[pl.ds(cb, NLANE)], idx, mask=mb)
            nt = plsc.all_reduce_population_count(mt)[0]
            nb = plsc.all_reduce_population_count(mb)[0]
            return ct + nt, cb + nb
        ct, cb = lax.fori_loop(0, CLEN, cpt, (jnp.int32(0), jnp.int32(0)))

        tk_v[pl.ds(ct, NLANE)] = neg_inf
        ti_v[pl.ds(ct, NLANE)] = jnp.zeros((NLANE,), jnp.int32)
        bk_v[pl.ds(cb, NLANE)] = pos_inf
        bi_v[pl.ds(cb, NLANE)] = jnp.zeros((NLANE,), jnp.int32)

        ak, ai = plsc.sort_key_val(tk_v[pl.ds(0, NLANE)], ti_v[pl.ds(0, NLANE)])

        def mrg_t(k, carry):
            ck, ci = carry
            sk, si = plsc.sort_key_val(tk_v[pl.ds(k * NLANE, NLANE)],
                                       ti_v[pl.ds(k * NLANE, NLANE)],
                                       descending=True)
            m = ck >= sk
            return tuple(plsc.sort_key_val(jnp.where(m, ck, sk),
                                           jnp.where(m, ci, si)))
        nch_t = (ct + NLANE - 1) // NLANE
        ak, ai = lax.fori_loop(1, nch_t, mrg_t, (ak, ai))
        _, fi = plsc.sort_key_val(ak, ai, descending=True)
        pos_v[r, :] = fi

        dk, di = plsc.sort_key_val(bk_v[pl.ds(0, NLANE)], bi_v[pl.ds(0, NLANE)],
                                   descending=True)

        def mrg_b(k, carry):
            ck, ci = carry
            sk, si = plsc.sort_key_val(bk_v[pl.ds(k * NLANE, NLANE)],
                                       bi_v[pl.ds(k * NLANE, NLANE)])
            m = ck <= sk
            return tuple(plsc.sort_key_val(jnp.where(m, ck, sk),
                                           jnp.where(m, ci, si),
                                           descending=True))
        nch_b = (cb + NLANE - 1) // NLANE
        dk, di = lax.fori_loop(1, nch_b, mrg_b, (dk, di))
        _, gi = plsc.sort_key_val(dk, di)
        neg_v[r, :] = gi
        return 0

    lax.fori_loop(0, rpw, row_body, 0)
    pltpu.sync_copy(pos_v, pos_hbm.at[pl.ds(base, rpw)])
    pltpu.sync_copy(neg_v, neg_hbm.at[pl.ds(base, rpw)])


@functools.cache
def _sc_topk(nrows):
    rpw = nrows // NW
    return functools.partial(
        pl.kernel,
        out_type=(jax.ShapeDtypeStruct((nrows, NLANE), jnp.int32),
                  jax.ShapeDtypeStruct((nrows, NLANE), jnp.int32)),
        mesh=plsc.VectorSubcoreMesh(core_axis_name="c", subcore_axis_name="s",
                                    num_cores=NC, num_subcores=NS),
        scratch_types=[
            pltpu.VMEM((rpw, NP), jnp.float32),
            pltpu.VMEM((rpw, NCHUNK), jnp.float32),
            pltpu.VMEM((rpw, NCHUNK), jnp.float32),
            pltpu.VMEM((CAND,), jnp.float32),
            pltpu.VMEM((CAND,), jnp.int32),
            pltpu.VMEM((CAND,), jnp.float32),
            pltpu.VMEM((CAND,), jnp.int32),
            pltpu.VMEM((rpw, NLANE), jnp.int32),
            pltpu.VMEM((rpw, NLANE), jnp.int32),
        ],
        compiler_params=pltpu.CompilerParams(needs_layout_passes=False),
    )(functools.partial(_sc_topk_body, rpw))


def kernel(basket_emb, item_emb, Wb, Wi, v):
    bp = basket_emb @ Wb.T
    ip = item_emb @ Wi.T
    bp, ip = lax.optimization_barrier((bp, ip))
    m = jnp.arange(NP)
    n_of_m = (m % NCHUNK) * CLEN + m // NCHUNK
    ip_pad = jnp.where((n_of_m < N)[:, None],
                       ip[jnp.minimum(n_of_m, N - 1)], 0.0)
    ipt = ip_pad.T
    vv = v.reshape(1, DIM)
    nsplit = 4
    part = B // nsplit
    tc = _tc_scores(part)
    sc = _sc_topk(part)
    tc_outs = [tc(bp[i * part:(i + 1) * part], ipt, vv) for i in range(nsplit)]
    sc_outs = [sc(*o) for o in tc_outs]
    pos = jnp.concatenate([o[0] for o in sc_outs], axis=0)
    neg = jnp.concatenate([o[1] for o in sc_outs], axis=0)
    return pos[:, :K], neg[:, :K]


# speedup vs baseline: 1.0860x; 1.0860x over previous
import functools

import jax
import jax.numpy as jnp
from jax import lax
from jax.experimental import pallas as pl
from jax.experimental.pallas import tpu as pltpu
from jax.experimental.pallas import tpu_sc as plsc

DIM = 16
K = 10
B = 1024
N = 1000
NP = 1024
NLANE = 16
NCHUNK = 16
CLEN = NP // NCHUNK
JVALID = N - (NCHUNK - 1) * CLEN
NC = 2
NS = 16
NW = NC * NS
RPW = B // NW
CAND = NP + NLANE
BLK = 128


def _tc_scores_body(bp_ref, ipt_ref, v_ref, out_ref, cmax_ref, cmin_ref):
    bp = bp_ref[...]
    ipt = ipt_ref[...]
    vv = v_ref[...].astype(jnp.bfloat16).astype(jnp.float32)
    acc = jnp.zeros((BLK, NP), jnp.float32)
    for d in range(DIM):
        h = jnp.tanh(bp[:, d:d + 1] + ipt[d:d + 1, :])
        hq = h.astype(jnp.bfloat16).astype(jnp.float32)
        acc = acc + vv[0:1, d:d + 1] * hq
    out_ref[...] = acc

    inv = (lax.broadcasted_iota(jnp.int32, (1, 128), 1) % NCHUNK) == NCHUNK - 1
    neg = jnp.float32(-jnp.inf)
    pos = jnp.float32(jnp.inf)
    cols_t = [acc[:, k * 128:(k + 1) * 128] for k in range(8)]
    cols_b = list(cols_t)
    for k in range(5, 8):
        cols_t[k] = jnp.where(inv, neg, cols_t[k])
        cols_b[k] = jnp.where(inv, pos, cols_b[k])
    mx = functools.reduce(jnp.maximum, cols_t)
    mn = functools.reduce(jnp.minimum, cols_b)
    for w in (64, 32, 16):
        mx = jnp.maximum(mx[:, :w], mx[:, w:2 * w])
        mn = jnp.minimum(mn[:, :w], mn[:, w:2 * w])
    cmax_ref[...] = mx
    cmin_ref[...] = mn


@functools.cache
def _tc_scores(nrows):
    return pl.pallas_call(
        _tc_scores_body,
        grid=(nrows // BLK,),
        in_specs=[
            pl.BlockSpec((BLK, DIM), lambda i: (i, 0)),
            pl.BlockSpec((DIM, NP), lambda i: (0, 0)),
            pl.BlockSpec((1, DIM), lambda i: (0, 0)),
        ],
        out_specs=[pl.BlockSpec((BLK, NP), lambda i: (i, 0)),
                   pl.BlockSpec((BLK, NCHUNK), lambda i: (i, 0)),
                   pl.BlockSpec((BLK, NCHUNK), lambda i: (i, 0))],
        out_shape=[jax.ShapeDtypeStruct((nrows, NP), jnp.float32),
                   jax.ShapeDtypeStruct((nrows, NCHUNK), jnp.float32),
                   jax.ShapeDtypeStruct((nrows, NCHUNK), jnp.float32)],
    )


def _sc_topk_body(rpw, scores_hbm, cmax_hbm, cmin_hbm, pos_hbm, neg_hbm,
                  rows_v, cmax_v, cmin_v, tk_v, ti_v, bk_v, bi_v, pos_v, neg_v):
    wid = lax.axis_index("s") * NC + lax.axis_index("c")
    base = wid * rpw
    pltpu.sync_copy(scores_hbm.at[pl.ds(base, rpw)], rows_v)
    pltpu.sync_copy(cmax_hbm.at[pl.ds(base, rpw)], cmax_v)
    pltpu.sync_copy(cmin_hbm.at[pl.ds(base, rpw)], cmin_v)

    lanes = lax.iota(jnp.int32, NLANE)
    lane_last = lanes == (NCHUNK - 1)
    neg_inf = jnp.full((NLANE,), -jnp.inf, jnp.float32)
    pos_inf = jnp.full((NLANE,), jnp.inf, jnp.float32)

    def row_body(r, _):
        tt = jnp.sort(cmax_v[r, :])[jnp.zeros((NLANE,), jnp.int32)]
        tb = jnp.sort(cmin_v[r, :])[jnp.full((NLANE,), NLANE - 1, jnp.int32)]

        def cpt(j, carry):
            ct, cb = carry
            vv = rows_v[r, pl.ds(j * NLANE, NLANE)]
            valid = jnp.logical_not(jnp.logical_and(lane_last, j >= JVALID))
            idx = lanes * CLEN + j
            mt = jnp.logical_and(vv >= tt, valid)
            mb = jnp.logical_and(vv <= tb, valid)
            plsc.store_compressed(tk_v.at[pl.ds(ct, NLANE)], vv, mask=mt)
            plsc.store_compressed(ti_v.at[pl.ds(ct, NLANE)], idx, mask=mt)
            plsc.store_compressed(bk_v.at[pl.ds(cb, NLANE)], vv, mask=mb)
            plsc.store_compressed(bi_v.at[pl.ds(cb, NLANE)], idx, mask=mb)
            nt = plsc.all_reduce_population_count(mt)[0]
            nb = plsc.all_reduce_population_count(mb)[0]
            return ct + nt, cb + nb
        ct, cb = lax.fori_loop(0, CLEN, cpt, (jnp.int32(0), jnp.int32(0)))

        tk_v[pl.ds(ct, NLANE)] = neg_inf
        ti_v[pl.ds(ct, NLANE)] = jnp.zeros((NLANE,), jnp.int32)
        bk_v[pl.ds(cb, NLANE)] = pos_inf
        bi_v[pl.ds(cb, NLANE)] = jnp.zeros((NLANE,), jnp.int32)

        ak, ai = plsc.sort_key_val(tk_v[pl.ds(0, NLANE)], ti_v[pl.ds(0, NLANE)])

        def mrg_t(k, carry):
            ck, ci = carry
            sk, si = plsc.sort_key_val(tk_v[pl.ds(k * NLANE, NLANE)],
                                       ti_v[pl.ds(k * NLANE, NLANE)],
                                       descending=True)
            m = ck >= sk
            return tuple(plsc.sort_key_val(jnp.where(m, ck, sk),
                                           jnp.where(m, ci, si)))
        nch_t = (ct + NLANE - 1) // NLANE
        ak, ai = lax.fori_loop(1, nch_t, mrg_t, (ak, ai))
        _, fi = plsc.sort_key_val(ak, ai, descending=True)
        pos_v[r, :] = fi

        dk, di = plsc.sort_key_val(bk_v[pl.ds(0, NLANE)], bi_v[pl.ds(0, NLANE)],
                                   descending=True)

        def mrg_b(k, carry):
            ck, ci = carry
            sk, si = plsc.sort_key_val(bk_v[pl.ds(k * NLANE, NLANE)],
                                       bi_v[pl.ds(k * NLANE, NLANE)])
            m = ck <= sk
            return tuple(plsc.sort_key_val(jnp.where(m, ck, sk),
                                           jnp.where(m, ci, si),
                                           descending=True))
        nch_b = (cb + NLANE - 1) // NLANE
        dk, di = lax.fori_loop(1, nch_b, mrg_b, (dk, di))
        _, gi = plsc.sort_key_val(dk, di)
        neg_v[r, :] = gi
        return 0

    lax.fori_loop(0, rpw, row_body, 0)
    pltpu.sync_copy(pos_v, pos_hbm.at[pl.ds(base, rpw)])
    pltpu.sync_copy(neg_v, neg_hbm.at[pl.ds(base, rpw)])


@functools.cache
def _sc_topk(nrows):
    rpw = nrows // NW
    return functools.partial(
        pl.kernel,
        out_type=(jax.ShapeDtypeStruct((nrows, NLANE), jnp.int32),
                  jax.ShapeDtypeStruct((nrows, NLANE), jnp.int32)),
        mesh=plsc.VectorSubcoreMesh(core_axis_name="c", subcore_axis_name="s",
                                    num_cores=NC, num_subcores=NS),
        scratch_types=[
            pltpu.VMEM((rpw, NP), jnp.float32),
            pltpu.VMEM((rpw, NCHUNK), jnp.float32),
            pltpu.VMEM((rpw, NCHUNK), jnp.float32),
            pltpu.VMEM((CAND,), jnp.float32),
            pltpu.VMEM((CAND,), jnp.int32),
            pltpu.VMEM((CAND,), jnp.float32),
            pltpu.VMEM((CAND,), jnp.int32),
            pltpu.VMEM((rpw, NLANE), jnp.int32),
            pltpu.VMEM((rpw, NLANE), jnp.int32),
        ],
        compiler_params=pltpu.CompilerParams(needs_layout_passes=False),
    )(functools.partial(_sc_topk_body, rpw))


def kernel(basket_emb, item_emb, Wb, Wi, v):
    bp = basket_emb @ Wb.T
    ip = item_emb @ Wi.T
    bp, ip = lax.optimization_barrier((bp, ip))
    m = jnp.arange(NP)
    n_of_m = (m % NCHUNK) * CLEN + m // NCHUNK
    ip_pad = jnp.where((n_of_m < N)[:, None],
                       ip[jnp.minimum(n_of_m, N - 1)], 0.0)
    ipt = ip_pad.T
    vv = v.reshape(1, DIM)
    nsplit = 2
    part = B // nsplit
    tc = _tc_scores(part)
    sc = _sc_topk(part)
    tc_outs = [tc(bp[i * part:(i + 1) * part], ipt, vv) for i in range(nsplit)]
    sc_outs = [sc(*o) for o in tc_outs]
    pos = jnp.concatenate([o[0] for o in sc_outs], axis=0)
    neg = jnp.concatenate([o[1] for o in sc_outs], axis=0)
    return pos[:, :K], neg[:, :K]
